# Initial kernel scaffold; baseline (speedup 1.0000x reference)
import functools

import jax
import jax.numpy as jnp
from jax import lax
from jax.experimental import pallas as pl
from jax.experimental.pallas import tpu as pltpu
from jax.experimental.pallas import tpu_sc as plsc

EMB = 128
V0, V1, V2 = 4, 2, 6
NCOMBO = V0 * V1 * V2
E = 320000
NC, NS = 2, 16
NW = NC * NS
SLAB = 128
NSLAB = E // SLAB
BASE = NSLAB // NW
EXTRA = NSLAB % NW
MAXSLABS = BASE + 1


def _bond_kernel(ea_hbm, w0_hbm, w1_hbm, w2_hbm, out_hbm, t_hbm,
                 ea_v, pk_v, rows_v, w0_v, w1_v, w2_v, t_v, gsem):
    cid = lax.axis_index("c")
    sid = lax.axis_index("s")
    wid = sid * NC + cid

    @pl.when(sid == 0)
    def _build_table():
        pltpu.sync_copy(w0_hbm, w0_v)
        pltpu.sync_copy(w1_hbm, w1_v)
        pltpu.sync_copy(w2_hbm, w2_v)
        for r in range(NCOMBO):
            i, j, k = r // (V1 * V2), (r // V2) % V1, r % V2

            def _g(g, carry, i=i, j=j, k=k, r=r):
                sl = pl.ds(g * 16, 16)
                t_v[r, sl] = w0_v[i, sl] + w1_v[j, sl] + w2_v[k, sl]
                return carry

            lax.fori_loop(0, EMB // 16, _g, 0)
        pltpu.sync_copy(t_v, t_hbm.at[pl.ds(cid * NCOMBO, NCOMBO)])

    plsc.subcore_barrier()

    start_slab = wid * BASE + jnp.maximum(wid - (NW - EXTRA), 0)
    ea_base = start_slab * SLAB * 3
    pltpu.sync_copy(ea_hbm.at[pl.ds(ea_base, MAXSLABS * SLAB * 3)], ea_v)

    lane = lax.iota(jnp.int32, 16)
    tbase = (cid * NCOMBO).astype(jnp.int32)

    def _pk(t, carry):
        e0 = (t * 16 + lane) * 3
        a = plsc.load_gather(ea_v, [e0])
        b = plsc.load_gather(ea_v, [e0 + 1])
        c = plsc.load_gather(ea_v, [e0 + 2])
        pk_v[pl.ds(t * 16, 16)] = a * (V1 * V2) + b * V2 + c + tbase
        return carry

    lax.fori_loop(0, MAXSLABS * SLAB // 16, _pk, 0)

    nslabs = jnp.where(wid >= NW - EXTRA, MAXSLABS, BASE)

    def _slab(t, carry):
        idx = pk_v.at[pl.ds(t * SLAB, SLAB)]
        pltpu.async_copy(t_hbm.at[idx], rows_v, gsem).wait()
        pltpu.sync_copy(rows_v, out_hbm.at[pl.ds((start_slab + t) * SLAB, SLAB)])
        return carry

    lax.fori_loop(0, nslabs, _slab, 0)


@jax.jit
def _run(ea_flat, w0, w1, w2):
    mesh = plsc.VectorSubcoreMesh(core_axis_name="c", subcore_axis_name="s",
                                  num_cores=NC, num_subcores=NS)
    out, _ = pl.kernel(
        _bond_kernel,
        out_type=(
            jax.ShapeDtypeStruct((E, EMB), jnp.float32),
            jax.ShapeDtypeStruct((NC * NCOMBO, EMB), jnp.float32),
        ),
        mesh=mesh,
        scratch_types=[
            pltpu.VMEM((MAXSLABS * SLAB * 3,), jnp.int32),
            pltpu.VMEM((MAXSLABS * SLAB,), jnp.int32),
            pltpu.VMEM((SLAB, EMB), jnp.float32),
            pltpu.VMEM((V0, EMB), jnp.float32),
            pltpu.VMEM((V1, EMB), jnp.float32),
            pltpu.VMEM((V2, EMB), jnp.float32),
            pltpu.VMEM((NCOMBO, EMB), jnp.float32),
            pltpu.SemaphoreType.DMA,
        ],
    )(ea_flat, w0, w1, w2)
    return out


def kernel(edge_attr, W0, W1, W2):
    ea_flat = edge_attr.astype(jnp.int32).reshape(-1)
    return _run(ea_flat, W0, W1, W2)


# speedup vs baseline: 1.2287x; 1.2287x over previous
import functools

import jax
import jax.numpy as jnp
from jax import lax
from jax.experimental import pallas as pl
from jax.experimental.pallas import tpu as pltpu
from jax.experimental.pallas import tpu_sc as plsc

EMB = 128
V0, V1, V2 = 4, 2, 6
NCOMBO = V0 * V1 * V2
E = 320000
NC, NS = 2, 16
NW = NC * NS
SLAB = 128
NSLAB = E // SLAB
BASE = NSLAB // NW
EXTRA = NSLAB % NW
MAXSLABS = BASE + 1


def _bond_kernel(e0_hbm, e1_hbm, e2_hbm, w0_hbm, w1_hbm, w2_hbm, out_hbm,
                 t_hbm, e0_v, e1_v, e2_v, pk_v, rows_v, w0_v, w1_v, w2_v,
                 t_v, gsem):
    cid = lax.axis_index("c")
    sid = lax.axis_index("s")
    wid = sid * NC + cid

    @pl.when(sid == 0)
    def _build_table():
        pltpu.sync_copy(w0_hbm, w0_v)
        pltpu.sync_copy(w1_hbm, w1_v)
        pltpu.sync_copy(w2_hbm, w2_v)
        for r in range(NCOMBO):
            i, j, k = r // (V1 * V2), (r // V2) % V1, r % V2

            def _g(g, carry, i=i, j=j, k=k, r=r):
                sl = pl.ds(g * 16, 16)
                t_v[r, sl] = w0_v[i, sl] + w1_v[j, sl] + w2_v[k, sl]
                return carry

            lax.fori_loop(0, EMB // 16, _g, 0)
        pltpu.sync_copy(t_v, t_hbm.at[pl.ds(cid * NCOMBO, NCOMBO)])

    plsc.subcore_barrier()

    start_slab = wid * BASE + jnp.maximum(wid - (NW - EXTRA), 0)
    ea_base = start_slab * SLAB
    pltpu.sync_copy(e0_hbm.at[pl.ds(ea_base, MAXSLABS * SLAB)], e0_v)
    pltpu.sync_copy(e1_hbm.at[pl.ds(ea_base, MAXSLABS * SLAB)], e1_v)
    pltpu.sync_copy(e2_hbm.at[pl.ds(ea_base, MAXSLABS * SLAB)], e2_v)

    tbase = (cid * NCOMBO).astype(jnp.int32)

    def _pk(t, carry):
        sl = pl.ds(t * 16, 16)
        pk_v[sl] = e0_v[sl] * (V1 * V2) + e1_v[sl] * V2 + e2_v[sl] + tbase
        return carry

    lax.fori_loop(0, MAXSLABS * SLAB // 16, _pk, 0)

    nslabs = jnp.where(wid >= NW - EXTRA, MAXSLABS, BASE)

    def _slab(t, carry):
        idx = pk_v.at[pl.ds(t * SLAB, SLAB)]
        pltpu.async_copy(t_hbm.at[idx], rows_v, gsem).wait()
        pltpu.sync_copy(rows_v, out_hbm.at[pl.ds((start_slab + t) * SLAB, SLAB)])
        return carry

    lax.fori_loop(0, nslabs, _slab, 0)


@jax.jit
def _run(e0, e1, e2, w0, w1, w2):
    mesh = plsc.VectorSubcoreMesh(core_axis_name="c", subcore_axis_name="s",
                                  num_cores=NC, num_subcores=NS)
    out, _ = pl.kernel(
        _bond_kernel,
        out_type=(
            jax.ShapeDtypeStruct((E, EMB), jnp.float32),
            jax.ShapeDtypeStruct((NC * NCOMBO, EMB), jnp.float32),
        ),
        mesh=mesh,
        scratch_types=[
            pltpu.VMEM((MAXSLABS * SLAB,), jnp.int32),
            pltpu.VMEM((MAXSLABS * SLAB,), jnp.int32),
            pltpu.VMEM((MAXSLABS * SLAB,), jnp.int32),
            pltpu.VMEM((MAXSLABS * SLAB,), jnp.int32),
            pltpu.VMEM((SLAB, EMB), jnp.float32),
            pltpu.VMEM((V0, EMB), jnp.float32),
            pltpu.VMEM((V1, EMB), jnp.float32),
            pltpu.VMEM((V2, EMB), jnp.float32),
            pltpu.VMEM((NCOMBO, EMB), jnp.float32),
            pltpu.SemaphoreType.DMA,
        ],
    )(e0, e1, e2, w0, w1, w2)
    return out


def kernel(edge_attr, W0, W1, W2):
    ea = edge_attr.astype(jnp.int32)
    return _run(ea[:, 0], ea[:, 1], ea[:, 2], W0, W1, W2)
